# baseline (speedup 1.0000x reference)
import functools

import jax
import jax.numpy as jnp
import numpy as np
from jax import lax
from jax.experimental import pallas as pl
from jax.experimental.pallas import tpu as pltpu
from jax.experimental.pallas import tpu_sc as plsc

_N = 10000
_E = 320000
_D = 128

_NC = 2
_NS = 16
_NW = _NC * _NS
_EPW = _E // _NW
_CHD = 40
_CWIN = 2000
_NWIN = _EPW // _CWIN
_WPAIR = _CWIN // (2 * _CHD)


def _mm_body(a_ref, w_ref, o_ref):
    o_ref[...] = jnp.dot(a_ref[...], w_ref[...],
                         preferred_element_type=jnp.float32
                         ).astype(jnp.bfloat16)


def _edge_mm(a, w, be=3200):
    e = a.shape[0]
    return pl.pallas_call(
        _mm_body,
        grid=(e // be,),
        in_specs=[pl.BlockSpec((be, _D), lambda i: (i, 0)),
                  pl.BlockSpec((_D, _D), lambda i: (0, 0))],
        out_specs=pl.BlockSpec((be, _D), lambda i: (i, 0)),
        out_shape=jax.ShapeDtypeStruct((e, _D), jnp.bfloat16),
    )(a, w)


def _to_words(m16):
    e = m16.shape[0]
    return lax.bitcast_convert_type(
        m16.reshape(e // 2, _D, 2), jnp.int32)


def _prep_body(we2_ref, we1e1_ref, be2_ref, be11_ref, wf_ref, c1_ref):
    wf_ref[...] = jnp.dot(we2_ref[...], we1e1_ref[...],
                          preferred_element_type=jnp.float32)
    c1_ref[...] = be11_ref[...] + jnp.dot(be2_ref[...], we1e1_ref[...],
                                          preferred_element_type=jnp.float32)


def _prep(we2_l0, we1e_l1, be2_l0, be1_l1):
    return pl.pallas_call(
        _prep_body,
        out_shape=[jax.ShapeDtypeStruct((_D, _D), jnp.float32),
                   jax.ShapeDtypeStruct((1, _D), jnp.float32)],
    )(we2_l0, we1e_l1, be2_l0.reshape(1, _D), be1_l1.reshape(1, _D))


def _nodepre_body(x_ref, ws_ref, wr_ref, b_ref, xs_ref, xr_ref):
    x = x_ref[...]
    xs_ref[...] = jnp.dot(x, ws_ref[...],
                          preferred_element_type=jnp.float32) + b_ref[...]
    xr_ref[...] = jnp.dot(x, wr_ref[...], preferred_element_type=jnp.float32)


def _nodepre(nodes, ws, wr, b, bn=1000):
    return pl.pallas_call(
        _nodepre_body,
        grid=(_N // bn,),
        in_specs=[pl.BlockSpec((bn, _D), lambda i: (i, 0)),
                  pl.BlockSpec((_D, _D), lambda i: (0, 0)),
                  pl.BlockSpec((_D, _D), lambda i: (0, 0)),
                  pl.BlockSpec((1, _D), lambda i: (0, 0))],
        out_specs=[pl.BlockSpec((bn, _D), lambda i: (i, 0)),
                   pl.BlockSpec((bn, _D), lambda i: (i, 0))],
        out_shape=[jax.ShapeDtypeStruct((_N, _D), jnp.float32),
                   jax.ShapeDtypeStruct((_N, _D), jnp.float32)],
    )(nodes, ws, wr, b.reshape(1, _D))


def _node_body(first, x_ref, sa_ref, sb_ref, ca_ref, cb_ref,
               we2_ref, be2_ref, wn1a_ref, wn1b_ref, bn1_ref, wn2_ref,
               bn2_ref, ws1_ref, wr1_ref, c1_ref, *out_refs):
    f32 = jnp.float32
    s = sa_ref[...] + sb_ref[...]
    cnt = (ca_ref[...] + cb_ref[...])[:, 0:1]
    eterm = jnp.dot(s, we2_ref[...], preferred_element_type=f32) \
        + cnt * be2_ref[...]
    agg = eterm / jnp.maximum(cnt, 1.0)
    hmid = jnp.maximum(
        jnp.dot(x_ref[...], wn1a_ref[...], preferred_element_type=f32)
        + jnp.dot(agg, wn1b_ref[...], preferred_element_type=f32)
        + bn1_ref[...], 0.0)
    newx = jnp.dot(hmid, wn2_ref[...], preferred_element_type=f32) \
        + bn2_ref[...]
    out_refs[0][...] = newx
    if first:
        out_refs[1][...] = jnp.dot(newx, ws1_ref[...],
                                   preferred_element_type=f32) + c1_ref[...]
        out_refs[2][...] = jnp.dot(newx, wr1_ref[...],
                                   preferred_element_type=f32)


def _node_layer(first, nodes, sa, sb, ca, cb, we2, be2, wn1a, wn1b, bn1,
                wn2, bn2, ws1, wr1, c1, bn=1000):
    blk = lambda r, c: pl.BlockSpec((r, c), lambda i: (i, 0))
    wspec = pl.BlockSpec((_D, _D), lambda i: (0, 0))
    bspec = pl.BlockSpec((1, _D), lambda i: (0, 0))
    nouts = 3 if first else 1
    outs = [jax.ShapeDtypeStruct((_N, _D), jnp.float32)] * nouts
    return pl.pallas_call(
        functools.partial(_node_body, first),
        grid=(_N // bn,),
        in_specs=[blk(bn, _D), blk(bn, _D), blk(bn, _D),
                  blk(bn, _D), blk(bn, _D),
                  wspec, bspec, wspec, wspec, bspec, wspec, bspec,
                  wspec, wspec, bspec],
        out_specs=[blk(bn, _D)] * nouts,
        out_shape=outs,
    )(nodes, sa, sb, ca, cb, we2, be2.reshape(1, _D), wn1a, wn1b,
      bn1.reshape(1, _D), wn2, bn2.reshape(1, _D), ws1, wr1, c1)


def _cnt_body(rcv_hbm, cz_hbm, cout_hbm, rall, ridxs0, ridxs1, ones, c_sh,
              ss0, ss1):
    c = lax.axis_index("c")
    s = lax.axis_index("s")
    wid = s * _NC + c
    base = wid * _EPW
    ridxs = (ridxs0, ridxs1)
    ssem = (ss0, ss1)

    @pl.when(s == 0)
    def _init():
        pltpu.sync_copy(cz_hbm, c_sh)

    lane = lax.iota(jnp.int32, 16)
    onev = jnp.where(lane == 0, 1.0, 0.0).astype(jnp.float32)
    zv = jnp.zeros((16,), jnp.float32)

    def initrow(r, carry):
        for v in range(_D // 16):
            ones[r, pl.ds(v * 16, 16)] = onev if v == 0 else zv
        return carry
    lax.fori_loop(0, _CHD, initrow, 0)
    plsc.subcore_barrier()

    def window(w, carry):
        pltpu.sync_copy(rcv_hbm.at[pl.ds(base + w * _CWIN, _CWIN)], rall)

        def pair(k, carry):
            j0 = 2 * k
            for b in (0, 1):
                o = (j0 + b) * _CHD

                @pl.when((k > 0) | (w > 0))
                def _():
                    pltpu.make_async_copy(ones, c_sh.at[ridxs[b]],
                                          ssem[b]).wait()
                for d in (0, 16, 24):
                    ridxs[b][pl.ds(d, 16)] = rall[pl.ds(o + d, 16)]
                pltpu.async_copy(ones, c_sh.at[ridxs[b]], ssem[b], add=True)
            return carry
        lax.fori_loop(0, _WPAIR, pair, 0)
        return carry
    lax.fori_loop(0, _NWIN, window, 0)

    pltpu.make_async_copy(ones, c_sh.at[ridxs[0]], ssem[0]).wait()
    pltpu.make_async_copy(ones, c_sh.at[ridxs[1]], ssem[1]).wait()
    plsc.subcore_barrier()

    @pl.when(s == 0)
    def _writeout():
        pltpu.sync_copy(c_sh, cout_hbm.at[c])


def _sc_cnt(rcv, czero):
    mesh = plsc.VectorSubcoreMesh(core_axis_name="c", subcore_axis_name="s")
    return pl.kernel(
        _cnt_body,
        out_type=[jax.ShapeDtypeStruct((_NC, _N, _D), jnp.float32)],
        mesh=mesh,
        scratch_types=[
            pltpu.VMEM((_CWIN,), jnp.int32),
            pltpu.VMEM((_CHD,), jnp.int32),
            pltpu.VMEM((_CHD,), jnp.int32),
            pltpu.VMEM((_CHD, _D), jnp.float32),
            pltpu.VMEM_SHARED((_N, _D), jnp.float32),
            pltpu.SemaphoreType.DMA,
            pltpu.SemaphoreType.DMA,
        ],
    )(rcv, czero)[0]


def _sc_body(first, m_hbm, xs_hbm, xr_hbm, snd_hbm, rcv_hbm, sz_hbm, *rest):
    if first:
        h_hbm, sout_hbm = rest[0], rest[1]
        scr = rest[2:]
    else:
        h_hbm = None
        sout_hbm = rest[0]
        scr = rest[1:]
    sall, rall = scr[0], scr[1]
    bufs = (scr[2:7], scr[7:12])
    sems = (scr[12:17], scr[17:22])
    s_sh = scr[22]
    isem = scr[23]
    mrows = (_CHD // 2 + 4, _CHD // 2 + 4)
    mdelta = (0, 4)

    c = lax.axis_index("c")
    s = lax.axis_index("s")
    wid = s * _NC + c
    base = wid * _EPW

    @pl.when(s == 0)
    def _init():
        pltpu.sync_copy(sz_hbm, s_sh)
    plsc.subcore_barrier()

    def issue_loads(w, t, b):
        (ridxs, xsb, xrb, mb, hb) = bufs[b]
        (gs, gr, gm, ss, sh) = sems[b]
        o = t * _CHD
        pltpu.async_copy(xs_hbm.at[sall.at[pl.ds(o, _CHD)]], xsb, gs)
        pltpu.async_copy(xr_hbm.at[rall.at[pl.ds(o, _CHD)]], xrb, gr)
        wo = wid * (_EPW // 2) + w * (_CWIN // 2) + (t // 2) * _CHD + 16 * b
        pltpu.async_copy(m_hbm.at[pl.ds(wo, mrows[b])], mb, gm)

    def drain_loads(b):
        (ridxs, xsb, xrb, mb, hb) = bufs[b]
        (gs, gr, gm, ss, sh) = sems[b]
        pltpu.make_async_copy(xs_hbm.at[sall.at[pl.ds(0, _CHD)]],
                              xsb, gs).wait()
        pltpu.make_async_copy(xr_hbm.at[rall.at[pl.ds(0, _CHD)]],
                              xrb, gr).wait()
        pltpu.make_async_copy(m_hbm.at[pl.ds(0, mrows[b])], mb, gm).wait()

    def drain_stores(b):
        (ridxs, xsb, xrb, mb, hb) = bufs[b]
        (gs, gr, gm, ss, sh) = sems[b]
        pltpu.make_async_copy(hb, s_sh.at[ridxs], ss).wait()
        if first:
            pltpu.make_async_copy(hb, h_hbm.at[pl.ds(0, _CHD)], sh).wait()

    def compute(b):
        (ridxs, xsb, xrb, mb, hb) = bufs[b]
        d = mdelta[b]
        himask = jnp.full((16,), -65536, jnp.int32)
        sh16 = jnp.full((16,), 16, jnp.int32)

        def wrow(q, carry):
            for p in range(2):
                r = 2 * q + p
                for g in range(_D // 32):
                    wv = mb[d + q, pl.ds(p * 64 + g * 16, 16)]
                    me = lax.bitcast_convert_type(wv << sh16, jnp.float32)
                    mo = lax.bitcast_convert_type(wv & himask, jnp.float32)
                    sle = pl.ds(g * 32, 16)
                    slo = pl.ds(g * 32 + 16, 16)
                    hb[r, sle] = jnp.maximum(
                        me + xsb[r, sle] + xrb[r, sle], 0.0)
                    hb[r, slo] = jnp.maximum(
                        mo + xsb[r, slo] + xrb[r, slo], 0.0)
            return carry
        lax.fori_loop(0, _CHD // 2, wrow, 0)

    def issue_stores(w, t, b):
        (ridxs, xsb, xrb, mb, hb) = bufs[b]
        (gs, gr, gm, ss, sh) = sems[b]
        o = t * _CHD
        for d in (0, 16, 24):
            ridxs[pl.ds(d, 16)] = rall[pl.ds(o + d, 16)]
        pltpu.async_copy(hb, s_sh.at[ridxs], ss, add=True)
        if first:
            pltpu.async_copy(hb, h_hbm.at[pl.ds(base + w * _CWIN + o, _CHD)],
                             sh)

    def window(w, carry):
        cps = pltpu.async_copy(
            snd_hbm.at[pl.ds(base + w * _CWIN, _CWIN)], sall, isem)
        pltpu.sync_copy(rcv_hbm.at[pl.ds(base + w * _CWIN, _CWIN)], rall)
        cps.wait()
        issue_loads(w, 0, 0)

        def pair(k, carry):
            j0 = 2 * k
            issue_loads(w, j0 + 1, 1)
            drain_loads(0)

            @pl.when((k > 0) | (w > 0))
            def _():
                drain_stores(0)
            compute(0)
            issue_stores(w, j0, 0)

            @pl.when(k < _WPAIR - 1)
            def _():
                issue_loads(w, j0 + 2, 0)
            drain_loads(1)

            @pl.when((k > 0) | (w > 0))
            def _():
                drain_stores(1)
            compute(1)
            issue_stores(w, j0 + 1, 1)
            return carry
        lax.fori_loop(0, _WPAIR, pair, 0)
        return carry
    lax.fori_loop(0, _NWIN, window, 0)

    drain_stores(0)
    drain_stores(1)
    plsc.subcore_barrier()

    @pl.when(s == 0)
    def _writeout():
        pltpu.sync_copy(s_sh, sout_hbm.at[c])


def _sc_edge(first, m, xs, xr, snd, rcv, szero):
    mesh = plsc.VectorSubcoreMesh(core_axis_name="c", subcore_axis_name="s")
    outs = [jax.ShapeDtypeStruct((_NC, _N, _D), jnp.float32)]
    if first:
        outs = [jax.ShapeDtypeStruct((_E, _D), jnp.float32)] + outs
    def bufset(mrow):
        return [
            pltpu.VMEM((_CHD,), jnp.int32),
            pltpu.VMEM((_CHD, _D), jnp.float32),
            pltpu.VMEM((_CHD, _D), jnp.float32),
            pltpu.VMEM((mrow, _D), jnp.int32),
            pltpu.VMEM((_CHD, _D), jnp.float32),
        ]
    semset = [pltpu.SemaphoreType.DMA] * 5
    scratch = [pltpu.VMEM((_CWIN,), jnp.int32),
               pltpu.VMEM((_CWIN,), jnp.int32)] \
        + bufset(_CHD // 2 + 4) + bufset(_CHD // 2 + 4) + semset + semset + [
        pltpu.VMEM_SHARED((_N, _D), jnp.float32),
        pltpu.SemaphoreType.DMA,
    ]
    fn = pl.kernel(
        functools.partial(_sc_body, first),
        out_type=outs,
        mesh=mesh,
        scratch_types=scratch,
    )
    return fn(m, xs, xr, snd, rcv, szero)


def kernel(nodes, edges, senders, receivers,
           l0_We1, l0_be1, l0_We2, l0_be2, l0_Wn1, l0_bn1, l0_Wn2, l0_bn2,
           l1_We1, l1_be1, l1_We2, l1_be2, l1_Wn1, l1_bn1, l1_Wn2, l1_bn2):
    snd = senders.astype(jnp.int32)
    rcv = receivers.astype(jnp.int32)

    we1e0, we1s0, we1r0 = l0_We1[:_D], l0_We1[_D:2 * _D], l0_We1[2 * _D:]
    we1e1, we1s1, we1r1 = l1_We1[:_D], l1_We1[_D:2 * _D], l1_We1[2 * _D:]
    wn1a0, wn1b0 = l0_Wn1[:_D], l0_Wn1[_D:]
    wn1a1, wn1b1 = l1_Wn1[:_D], l1_Wn1[_D:]

    szero = jnp.zeros((_N, _D), jnp.float32)

    lanes = np.arange(_D)
    g, k = lanes // 32, lanes % 32
    sigma = 32 * g + np.where(k < 16, 2 * k, 2 * (k - 16) + 1)

    wf, c1 = _prep(l0_We2[sigma], we1e1, l0_be2, l1_be1)

    cnt2 = _sc_cnt(rcv, szero)

    xs0, xr0 = _nodepre(nodes, we1s0[:, sigma], we1r0[:, sigma],
                        l0_be1[sigma])
    m0 = _to_words(_edge_mm(edges, we1e0))
    h0, s0 = _sc_edge(True, m0, xs0, xr0, snd, rcv, szero)
    nodes1, xs1, xr1 = _node_layer(
        True, nodes, s0[0], s0[1], cnt2[0], cnt2[1],
        l0_We2[sigma], l0_be2, wn1a0, wn1b0, l0_bn1, l0_Wn2, l0_bn2,
        we1s1[:, sigma], we1r1[:, sigma], c1[:, sigma])

    m1 = _to_words(_edge_mm(h0, wf))
    (s1,) = _sc_edge(False, m1, xs1, xr1, snd, rcv, szero)
    (nodes2,) = _node_layer(
        False, nodes1, s1[0], s1[1], cnt2[0], cnt2[1],
        l1_We2[sigma], l1_be2, wn1a1, wn1b1, l1_bn1, l1_Wn2, l1_bn2,
        jnp.zeros((_D, _D), jnp.float32), jnp.zeros((_D, _D), jnp.float32),
        c1)
    return nodes2


# speedup vs baseline: 31.3604x; 31.3604x over previous
import functools

import jax
import jax.numpy as jnp
from jax import lax
from jax.experimental import pallas as pl
from jax.experimental.pallas import tpu as pltpu
from jax.experimental.pallas import tpu_sc as plsc

_N = 10000
_E = 320000
_D = 128

_NC = 2
_NS = 16
_NW = _NC * _NS
_EPW = _E // _NW
_CHD = 40
_CWIN = 2000
_NWIN = _EPW // _CWIN
_WPAIR = _CWIN // (2 * _CHD)


def _mm_body(a_ref, w_ref, o_ref):
    o_ref[...] = jnp.dot(a_ref[...], w_ref[...],
                         preferred_element_type=jnp.float32)


def _edge_mm(a, w, be=3200):
    e = a.shape[0]
    return pl.pallas_call(
        _mm_body,
        grid=(e // be,),
        in_specs=[pl.BlockSpec((be, _D), lambda i: (i, 0)),
                  pl.BlockSpec((_D, _D), lambda i: (0, 0))],
        out_specs=pl.BlockSpec((be, _D), lambda i: (i, 0)),
        out_shape=jax.ShapeDtypeStruct((e, _D), jnp.float32),
    )(a, w)


def _prep_body(we2_ref, we1e1_ref, be2_ref, be11_ref, wf_ref, c1_ref):
    wf_ref[...] = jnp.dot(we2_ref[...], we1e1_ref[...],
                          preferred_element_type=jnp.float32)
    c1_ref[...] = be11_ref[...] + jnp.dot(be2_ref[...], we1e1_ref[...],
                                          preferred_element_type=jnp.float32)


def _prep(we2_l0, we1e_l1, be2_l0, be1_l1):
    return pl.pallas_call(
        _prep_body,
        out_shape=[jax.ShapeDtypeStruct((_D, _D), jnp.float32),
                   jax.ShapeDtypeStruct((1, _D), jnp.float32)],
    )(we2_l0, we1e_l1, be2_l0.reshape(1, _D), be1_l1.reshape(1, _D))


def _nodepre_body(x_ref, ws_ref, wr_ref, b_ref, xs_ref, xr_ref):
    x = x_ref[...]
    xs_ref[...] = jnp.dot(x, ws_ref[...],
                          preferred_element_type=jnp.float32) + b_ref[...]
    xr_ref[...] = jnp.dot(x, wr_ref[...], preferred_element_type=jnp.float32)


def _nodepre(nodes, ws, wr, b, bn=1000):
    return pl.pallas_call(
        _nodepre_body,
        grid=(_N // bn,),
        in_specs=[pl.BlockSpec((bn, _D), lambda i: (i, 0)),
                  pl.BlockSpec((_D, _D), lambda i: (0, 0)),
                  pl.BlockSpec((_D, _D), lambda i: (0, 0)),
                  pl.BlockSpec((1, _D), lambda i: (0, 0))],
        out_specs=[pl.BlockSpec((bn, _D), lambda i: (i, 0)),
                   pl.BlockSpec((bn, _D), lambda i: (i, 0))],
        out_shape=[jax.ShapeDtypeStruct((_N, _D), jnp.float32),
                   jax.ShapeDtypeStruct((_N, _D), jnp.float32)],
    )(nodes, ws, wr, b.reshape(1, _D))


def _node_body(first, x_ref, sa_ref, sb_ref, ca_ref, cb_ref,
               we2_ref, be2_ref, wn1a_ref, wn1b_ref, bn1_ref, wn2_ref,
               bn2_ref, ws1_ref, wr1_ref, c1_ref, *out_refs):
    f32 = jnp.float32
    s = sa_ref[...] + sb_ref[...]
    cnt = (ca_ref[...] + cb_ref[...])[:, 0:1]
    eterm = jnp.dot(s, we2_ref[...], preferred_element_type=f32) \
        + cnt * be2_ref[...]
    agg = eterm / jnp.maximum(cnt, 1.0)
    hmid = jnp.maximum(
        jnp.dot(x_ref[...], wn1a_ref[...], preferred_element_type=f32)
        + jnp.dot(agg, wn1b_ref[...], preferred_element_type=f32)
        + bn1_ref[...], 0.0)
    newx = jnp.dot(hmid, wn2_ref[...], preferred_element_type=f32) \
        + bn2_ref[...]
    out_refs[0][...] = newx
    if first:
        out_refs[1][...] = jnp.dot(newx, ws1_ref[...],
                                   preferred_element_type=f32) + c1_ref[...]
        out_refs[2][...] = jnp.dot(newx, wr1_ref[...],
                                   preferred_element_type=f32)


def _node_layer(first, nodes, sa, sb, ca, cb, we2, be2, wn1a, wn1b, bn1,
                wn2, bn2, ws1, wr1, c1, bn=1000):
    blk = lambda r, c: pl.BlockSpec((r, c), lambda i: (i, 0))
    wspec = pl.BlockSpec((_D, _D), lambda i: (0, 0))
    bspec = pl.BlockSpec((1, _D), lambda i: (0, 0))
    nouts = 3 if first else 1
    outs = [jax.ShapeDtypeStruct((_N, _D), jnp.float32)] * nouts
    return pl.pallas_call(
        functools.partial(_node_body, first),
        grid=(_N // bn,),
        in_specs=[blk(bn, _D), blk(bn, _D), blk(bn, _D),
                  blk(bn, _D), blk(bn, _D),
                  wspec, bspec, wspec, wspec, bspec, wspec, bspec,
                  wspec, wspec, bspec],
        out_specs=[blk(bn, _D)] * nouts,
        out_shape=outs,
    )(nodes, sa, sb, ca, cb, we2, be2.reshape(1, _D), wn1a, wn1b,
      bn1.reshape(1, _D), wn2, bn2.reshape(1, _D), ws1, wr1, c1)


def _cnt_body(rcv_hbm, cz_hbm, cout_hbm, rall, ridxs0, ridxs1, ones, c_sh,
              ss0, ss1):
    c = lax.axis_index("c")
    s = lax.axis_index("s")
    wid = s * _NC + c
    base = wid * _EPW
    ridxs = (ridxs0, ridxs1)
    ssem = (ss0, ss1)

    @pl.when(s == 0)
    def _init():
        pltpu.sync_copy(cz_hbm, c_sh)

    lane = lax.iota(jnp.int32, 16)
    onev = jnp.where(lane == 0, 1.0, 0.0).astype(jnp.float32)
    zv = jnp.zeros((16,), jnp.float32)

    def initrow(r, carry):
        for v in range(_D // 16):
            ones[r, pl.ds(v * 16, 16)] = onev if v == 0 else zv
        return carry
    lax.fori_loop(0, _CHD, initrow, 0)
    plsc.subcore_barrier()

    def window(w, carry):
        pltpu.sync_copy(rcv_hbm.at[pl.ds(base + w * _CWIN, _CWIN)], rall)

        def pair(k, carry):
            j0 = 2 * k
            for b in (0, 1):
                o = (j0 + b) * _CHD

                @pl.when((k > 0) | (w > 0))
                def _():
                    pltpu.make_async_copy(ones, c_sh.at[ridxs[b]],
                                          ssem[b]).wait()
                for d in (0, 16, 24):
                    ridxs[b][pl.ds(d, 16)] = rall[pl.ds(o + d, 16)]
                pltpu.async_copy(ones, c_sh.at[ridxs[b]], ssem[b], add=True)
            return carry
        lax.fori_loop(0, _WPAIR, pair, 0)
        return carry
    lax.fori_loop(0, _NWIN, window, 0)

    pltpu.make_async_copy(ones, c_sh.at[ridxs[0]], ssem[0]).wait()
    pltpu.make_async_copy(ones, c_sh.at[ridxs[1]], ssem[1]).wait()
    plsc.subcore_barrier()

    @pl.when(s == 0)
    def _writeout():
        pltpu.sync_copy(c_sh, cout_hbm.at[c])


def _sc_cnt(rcv, czero):
    mesh = plsc.VectorSubcoreMesh(core_axis_name="c", subcore_axis_name="s")
    return pl.kernel(
        _cnt_body,
        out_type=[jax.ShapeDtypeStruct((_NC, _N, _D), jnp.float32)],
        mesh=mesh,
        scratch_types=[
            pltpu.VMEM((_CWIN,), jnp.int32),
            pltpu.VMEM((_CHD,), jnp.int32),
            pltpu.VMEM((_CHD,), jnp.int32),
            pltpu.VMEM((_CHD, _D), jnp.float32),
            pltpu.VMEM_SHARED((_N, _D), jnp.float32),
            pltpu.SemaphoreType.DMA,
            pltpu.SemaphoreType.DMA,
        ],
    )(rcv, czero)[0]


def _sc_body(first, m_hbm, xs_hbm, xr_hbm, snd_hbm, rcv_hbm, sz_hbm, *rest):
    if first:
        h_hbm, sout_hbm = rest[0], rest[1]
        scr = rest[2:]
    else:
        h_hbm = None
        sout_hbm = rest[0]
        scr = rest[1:]
    sall, rall = scr[0], scr[1]
    bufs = (scr[2:7], scr[7:12])
    sems = (scr[12:17], scr[17:22])
    s_sh = scr[22]
    isem = scr[23]

    c = lax.axis_index("c")
    s = lax.axis_index("s")
    wid = s * _NC + c
    base = wid * _EPW

    @pl.when(s == 0)
    def _init():
        pltpu.sync_copy(sz_hbm, s_sh)
    plsc.subcore_barrier()

    def issue_loads(w, t, b):
        (ridxs, xsb, xrb, mb, hb) = bufs[b]
        (gs, gr, gm, ss, sh) = sems[b]
        o = t * _CHD
        pltpu.async_copy(xs_hbm.at[sall.at[pl.ds(o, _CHD)]], xsb, gs)
        pltpu.async_copy(xr_hbm.at[rall.at[pl.ds(o, _CHD)]], xrb, gr)
        pltpu.async_copy(m_hbm.at[pl.ds(base + w * _CWIN + o, _CHD)],
                         mb, gm)

    def drain_loads(b):
        (ridxs, xsb, xrb, mb, hb) = bufs[b]
        (gs, gr, gm, ss, sh) = sems[b]
        pltpu.make_async_copy(xs_hbm.at[sall.at[pl.ds(0, _CHD)]],
                              xsb, gs).wait()
        pltpu.make_async_copy(xr_hbm.at[rall.at[pl.ds(0, _CHD)]],
                              xrb, gr).wait()
        pltpu.make_async_copy(m_hbm.at[pl.ds(0, _CHD)], mb, gm).wait()

    def drain_stores(b):
        (ridxs, xsb, xrb, mb, hb) = bufs[b]
        (gs, gr, gm, ss, sh) = sems[b]
        pltpu.make_async_copy(hb, s_sh.at[ridxs], ss).wait()
        if first:
            pltpu.make_async_copy(hb, h_hbm.at[pl.ds(0, _CHD)], sh).wait()

    def compute(b):
        (ridxs, xsb, xrb, mb, hb) = bufs[b]

        def row(r, carry):
            for v in range(_D // 16):
                sl = pl.ds(v * 16, 16)
                hv = jnp.maximum(mb[r, sl] + xsb[r, sl] + xrb[r, sl], 0.0)
                hb[r, sl] = hv
            return carry
        lax.fori_loop(0, _CHD, row, 0)

    def issue_stores(w, t, b):
        (ridxs, xsb, xrb, mb, hb) = bufs[b]
        (gs, gr, gm, ss, sh) = sems[b]
        o = t * _CHD
        for d in (0, 16, 24):
            ridxs[pl.ds(d, 16)] = rall[pl.ds(o + d, 16)]
        pltpu.async_copy(hb, s_sh.at[ridxs], ss, add=True)
        if first:
            pltpu.async_copy(hb, h_hbm.at[pl.ds(base + w * _CWIN + o, _CHD)],
                             sh)

    def window(w, carry):
        cps = pltpu.async_copy(
            snd_hbm.at[pl.ds(base + w * _CWIN, _CWIN)], sall, isem)
        pltpu.sync_copy(rcv_hbm.at[pl.ds(base + w * _CWIN, _CWIN)], rall)
        cps.wait()
        issue_loads(w, 0, 0)

        def pair(k, carry):
            j0 = 2 * k
            issue_loads(w, j0 + 1, 1)
            drain_loads(0)

            @pl.when((k > 0) | (w > 0))
            def _():
                drain_stores(0)
            compute(0)
            issue_stores(w, j0, 0)

            @pl.when(k < _WPAIR - 1)
            def _():
                issue_loads(w, j0 + 2, 0)
            drain_loads(1)

            @pl.when((k > 0) | (w > 0))
            def _():
                drain_stores(1)
            compute(1)
            issue_stores(w, j0 + 1, 1)
            return carry
        lax.fori_loop(0, _WPAIR, pair, 0)
        return carry
    lax.fori_loop(0, _NWIN, window, 0)

    drain_stores(0)
    drain_stores(1)
    plsc.subcore_barrier()

    @pl.when(s == 0)
    def _writeout():
        pltpu.sync_copy(s_sh, sout_hbm.at[c])


def _sc_edge(first, m, xs, xr, snd, rcv, szero):
    mesh = plsc.VectorSubcoreMesh(core_axis_name="c", subcore_axis_name="s")
    outs = [jax.ShapeDtypeStruct((_NC, _N, _D), jnp.float32)]
    if first:
        outs = [jax.ShapeDtypeStruct((_E, _D), jnp.float32)] + outs
    bufset = [
        pltpu.VMEM((_CHD,), jnp.int32),
        pltpu.VMEM((_CHD, _D), jnp.float32),
        pltpu.VMEM((_CHD, _D), jnp.float32),
        pltpu.VMEM((_CHD, _D), jnp.float32),
        pltpu.VMEM((_CHD, _D), jnp.float32),
    ]
    semset = [pltpu.SemaphoreType.DMA] * 5
    scratch = [pltpu.VMEM((_CWIN,), jnp.int32),
               pltpu.VMEM((_CWIN,), jnp.int32)] \
        + bufset + bufset + semset + semset + [
        pltpu.VMEM_SHARED((_N, _D), jnp.float32),
        pltpu.SemaphoreType.DMA,
    ]
    fn = pl.kernel(
        functools.partial(_sc_body, first),
        out_type=outs,
        mesh=mesh,
        scratch_types=scratch,
    )
    return fn(m, xs, xr, snd, rcv, szero)


def kernel(nodes, edges, senders, receivers,
           l0_We1, l0_be1, l0_We2, l0_be2, l0_Wn1, l0_bn1, l0_Wn2, l0_bn2,
           l1_We1, l1_be1, l1_We2, l1_be2, l1_Wn1, l1_bn1, l1_Wn2, l1_bn2):
    snd = senders.astype(jnp.int32)
    rcv = receivers.astype(jnp.int32)

    we1e0, we1s0, we1r0 = l0_We1[:_D], l0_We1[_D:2 * _D], l0_We1[2 * _D:]
    we1e1, we1s1, we1r1 = l1_We1[:_D], l1_We1[_D:2 * _D], l1_We1[2 * _D:]
    wn1a0, wn1b0 = l0_Wn1[:_D], l0_Wn1[_D:]
    wn1a1, wn1b1 = l1_Wn1[:_D], l1_Wn1[_D:]

    szero = jnp.zeros((_N, _D), jnp.float32)

    wf, c1 = _prep(l0_We2, we1e1, l0_be2, l1_be1)

    cnt2 = _sc_cnt(rcv, szero)

    xs0, xr0 = _nodepre(nodes, we1s0, we1r0, l0_be1)
    m0 = _edge_mm(edges, we1e0)
    h0, s0 = _sc_edge(True, m0, xs0, xr0, snd, rcv, szero)
    nodes1, xs1, xr1 = _node_layer(
        True, nodes, s0[0], s0[1], cnt2[0], cnt2[1],
        l0_We2, l0_be2, wn1a0, wn1b0, l0_bn1, l0_Wn2, l0_bn2,
        we1s1, we1r1, c1)

    m1 = _edge_mm(h0, wf)
    (s1,) = _sc_edge(False, m1, xs1, xr1, snd, rcv, szero)
    (nodes2,) = _node_layer(
        False, nodes1, s1[0], s1[1], cnt2[0], cnt2[1],
        l1_We2, l1_be2, wn1a1, wn1b1, l1_bn1, l1_Wn2, l1_bn2,
        jnp.zeros((_D, _D), jnp.float32), jnp.zeros((_D, _D), jnp.float32),
        c1)
    return nodes2


# speedup vs baseline: 33.4274x; 1.0659x over previous
import functools

import jax
import jax.numpy as jnp
from jax import lax
from jax.experimental import pallas as pl
from jax.experimental.pallas import tpu as pltpu
from jax.experimental.pallas import tpu_sc as plsc

_N = 10000
_E = 320000
_D = 128

_NC = 2
_NS = 16
_NW = _NC * _NS
_EPW = _E // _NW
_CHD = 40
_CWIN = 2000
_NWIN = _EPW // _CWIN
_WPAIR = _CWIN // (2 * _CHD)


def _mm_body(a_ref, w_ref, o_ref):
    o_ref[...] = jnp.dot(a_ref[...], w_ref[...],
                         preferred_element_type=jnp.float32)


def _edge_mm(a, w, be=8000):
    e = a.shape[0]
    return pl.pallas_call(
        _mm_body,
        grid=(e // be,),
        in_specs=[pl.BlockSpec((be, _D), lambda i: (i, 0)),
                  pl.BlockSpec((_D, _D), lambda i: (0, 0))],
        out_specs=pl.BlockSpec((be, _D), lambda i: (i, 0)),
        out_shape=jax.ShapeDtypeStruct((e, _D), jnp.float32),
    )(a, w)


def _prep_body(we2_ref, we1e1_ref, be2_ref, be11_ref, wf_ref, c1_ref):
    wf_ref[...] = jnp.dot(we2_ref[...], we1e1_ref[...],
                          preferred_element_type=jnp.float32)
    c1_ref[...] = be11_ref[...] + jnp.dot(be2_ref[...], we1e1_ref[...],
                                          preferred_element_type=jnp.float32)


def _prep(we2_l0, we1e_l1, be2_l0, be1_l1):
    return pl.pallas_call(
        _prep_body,
        out_shape=[jax.ShapeDtypeStruct((_D, _D), jnp.float32),
                   jax.ShapeDtypeStruct((1, _D), jnp.float32)],
    )(we2_l0, we1e_l1, be2_l0.reshape(1, _D), be1_l1.reshape(1, _D))


def _nodepre_body(x_ref, ws_ref, wr_ref, b_ref, xs_ref, xr_ref):
    x = x_ref[...]
    xs_ref[...] = jnp.dot(x, ws_ref[...],
                          preferred_element_type=jnp.float32) + b_ref[...]
    xr_ref[...] = jnp.dot(x, wr_ref[...], preferred_element_type=jnp.float32)


def _nodepre(nodes, ws, wr, b, bn=1000):
    return pl.pallas_call(
        _nodepre_body,
        grid=(_N // bn,),
        in_specs=[pl.BlockSpec((bn, _D), lambda i: (i, 0)),
                  pl.BlockSpec((_D, _D), lambda i: (0, 0)),
                  pl.BlockSpec((_D, _D), lambda i: (0, 0)),
                  pl.BlockSpec((1, _D), lambda i: (0, 0))],
        out_specs=[pl.BlockSpec((bn, _D), lambda i: (i, 0)),
                   pl.BlockSpec((bn, _D), lambda i: (i, 0))],
        out_shape=[jax.ShapeDtypeStruct((_N, _D), jnp.float32),
                   jax.ShapeDtypeStruct((_N, _D), jnp.float32)],
    )(nodes, ws, wr, b.reshape(1, _D))


def _node_body(first, x_ref, sa_ref, sb_ref, ca_ref, cb_ref,
               we2_ref, be2_ref, wn1a_ref, wn1b_ref, bn1_ref, wn2_ref,
               bn2_ref, ws1_ref, wr1_ref, c1_ref, *out_refs):
    f32 = jnp.float32
    s = sa_ref[...] + sb_ref[...]
    cnt = (ca_ref[...] + cb_ref[...])[:, 0:1]
    eterm = jnp.dot(s, we2_ref[...], preferred_element_type=f32) \
        + cnt * be2_ref[...]
    agg = eterm / jnp.maximum(cnt, 1.0)
    hmid = jnp.maximum(
        jnp.dot(x_ref[...], wn1a_ref[...], preferred_element_type=f32)
        + jnp.dot(agg, wn1b_ref[...], preferred_element_type=f32)
        + bn1_ref[...], 0.0)
    newx = jnp.dot(hmid, wn2_ref[...], preferred_element_type=f32) \
        + bn2_ref[...]
    out_refs[0][...] = newx
    if first:
        out_refs[1][...] = jnp.dot(newx, ws1_ref[...],
                                   preferred_element_type=f32) + c1_ref[...]
        out_refs[2][...] = jnp.dot(newx, wr1_ref[...],
                                   preferred_element_type=f32)


def _node_layer(first, nodes, sa, sb, ca, cb, we2, be2, wn1a, wn1b, bn1,
                wn2, bn2, ws1, wr1, c1, bn=2000):
    blk = lambda r, c: pl.BlockSpec((r, c), lambda i: (i, 0))
    wspec = pl.BlockSpec((_D, _D), lambda i: (0, 0))
    bspec = pl.BlockSpec((1, _D), lambda i: (0, 0))
    nouts = 3 if first else 1
    outs = [jax.ShapeDtypeStruct((_N, _D), jnp.float32)] * nouts
    return pl.pallas_call(
        functools.partial(_node_body, first),
        grid=(_N // bn,),
        in_specs=[blk(bn, _D), blk(bn, _D), blk(bn, _D),
                  blk(bn, _D), blk(bn, _D),
                  wspec, bspec, wspec, wspec, bspec, wspec, bspec,
                  wspec, wspec, bspec],
        out_specs=[blk(bn, _D)] * nouts,
        out_shape=outs,
    )(nodes, sa, sb, ca, cb, we2, be2.reshape(1, _D), wn1a, wn1b,
      bn1.reshape(1, _D), wn2, bn2.reshape(1, _D), ws1, wr1, c1)


def _cnt_body(rcv_hbm, cz_hbm, cout_hbm, rall, ridxs0, ridxs1, ones, c_sh,
              ss0, ss1):
    c = lax.axis_index("c")
    s = lax.axis_index("s")
    wid = s * _NC + c
    base = wid * _EPW
    ridxs = (ridxs0, ridxs1)
    ssem = (ss0, ss1)

    @pl.when(s == 0)
    def _init():
        pltpu.sync_copy(cz_hbm, c_sh)

    lane = lax.iota(jnp.int32, 16)
    onev = jnp.where(lane == 0, 1.0, 0.0).astype(jnp.float32)
    zv = jnp.zeros((16,), jnp.float32)

    def initrow(r, carry):
        for v in range(_D // 16):
            ones[r, pl.ds(v * 16, 16)] = onev if v == 0 else zv
        return carry
    lax.fori_loop(0, _CHD, initrow, 0)
    plsc.subcore_barrier()

    def window(w, carry):
        pltpu.sync_copy(rcv_hbm.at[pl.ds(base + w * _CWIN, _CWIN)], rall)

        def pair(k, carry):
            j0 = 2 * k
            for b in (0, 1):
                o = (j0 + b) * _CHD

                @pl.when((k > 0) | (w > 0))
                def _():
                    pltpu.make_async_copy(ones, c_sh.at[ridxs[b]],
                                          ssem[b]).wait()
                for d in (0, 16, 24):
                    ridxs[b][pl.ds(d, 16)] = rall[pl.ds(o + d, 16)]
                pltpu.async_copy(ones, c_sh.at[ridxs[b]], ssem[b], add=True)
            return carry
        lax.fori_loop(0, _WPAIR, pair, 0)
        return carry
    lax.fori_loop(0, _NWIN, window, 0)

    pltpu.make_async_copy(ones, c_sh.at[ridxs[0]], ssem[0]).wait()
    pltpu.make_async_copy(ones, c_sh.at[ridxs[1]], ssem[1]).wait()
    plsc.subcore_barrier()

    @pl.when(s == 0)
    def _writeout():
        pltpu.sync_copy(c_sh, cout_hbm.at[c])


def _sc_cnt(rcv, czero):
    mesh = plsc.VectorSubcoreMesh(core_axis_name="c", subcore_axis_name="s")
    return pl.kernel(
        _cnt_body,
        out_type=[jax.ShapeDtypeStruct((_NC, _N, _D), jnp.float32)],
        mesh=mesh,
        scratch_types=[
            pltpu.VMEM((_CWIN,), jnp.int32),
            pltpu.VMEM((_CHD,), jnp.int32),
            pltpu.VMEM((_CHD,), jnp.int32),
            pltpu.VMEM((_CHD, _D), jnp.float32),
            pltpu.VMEM_SHARED((_N, _D), jnp.float32),
            pltpu.SemaphoreType.DMA,
            pltpu.SemaphoreType.DMA,
        ],
    )(rcv, czero)[0]


def _sc_body(first, m_hbm, xs_hbm, xr_hbm, snd_hbm, rcv_hbm, sz_hbm, *rest):
    if first:
        h_hbm, sout_hbm = rest[0], rest[1]
        scr = rest[2:]
    else:
        h_hbm = None
        sout_hbm = rest[0]
        scr = rest[1:]
    sall, rall = scr[0], scr[1]
    bufs = (scr[2:7], scr[7:12])
    sems = (scr[12:17], scr[17:22])
    s_sh = scr[22]
    isem = scr[23]

    c = lax.axis_index("c")
    s = lax.axis_index("s")
    wid = s * _NC + c
    base = wid * _EPW

    @pl.when(s == 0)
    def _init():
        pltpu.sync_copy(sz_hbm, s_sh)
    plsc.subcore_barrier()

    def issue_loads(w, t, b):
        (ridxs, xsb, xrb, mb, hb) = bufs[b]
        (gs, gr, gm, ss, sh) = sems[b]
        o = t * _CHD
        pltpu.async_copy(xs_hbm.at[sall.at[pl.ds(o, _CHD)]], xsb, gs)
        pltpu.async_copy(xr_hbm.at[rall.at[pl.ds(o, _CHD)]], xrb, gr)
        pltpu.async_copy(m_hbm.at[pl.ds(base + w * _CWIN + o, _CHD)],
                         mb, gm)

    def drain_loads(b):
        (ridxs, xsb, xrb, mb, hb) = bufs[b]
        (gs, gr, gm, ss, sh) = sems[b]
        pltpu.make_async_copy(xs_hbm.at[sall.at[pl.ds(0, _CHD)]],
                              xsb, gs).wait()
        pltpu.make_async_copy(xr_hbm.at[rall.at[pl.ds(0, _CHD)]],
                              xrb, gr).wait()
        pltpu.make_async_copy(m_hbm.at[pl.ds(0, _CHD)], mb, gm).wait()

    def drain_stores(b):
        (ridxs, xsb, xrb, mb, hb) = bufs[b]
        (gs, gr, gm, ss, sh) = sems[b]
        pltpu.make_async_copy(hb, s_sh.at[ridxs], ss).wait()
        if first:
            pltpu.make_async_copy(hb, h_hbm.at[pl.ds(0, _CHD)], sh).wait()

    def compute(b):
        (ridxs, xsb, xrb, mb, hb) = bufs[b]

        def row(r, carry):
            for v in range(_D // 16):
                sl = pl.ds(v * 16, 16)
                hv = jnp.maximum(mb[r, sl] + xsb[r, sl] + xrb[r, sl], 0.0)
                hb[r, sl] = hv
            return carry
        lax.fori_loop(0, _CHD, row, 0)

    def issue_stores(w, t, b):
        (ridxs, xsb, xrb, mb, hb) = bufs[b]
        (gs, gr, gm, ss, sh) = sems[b]
        o = t * _CHD
        for d in (0, 16, 24):
            ridxs[pl.ds(d, 16)] = rall[pl.ds(o + d, 16)]
        pltpu.async_copy(hb, s_sh.at[ridxs], ss, add=True)
        if first:
            pltpu.async_copy(hb, h_hbm.at[pl.ds(base + w * _CWIN + o, _CHD)],
                             sh)

    def window(w, carry):
        cps = pltpu.async_copy(
            snd_hbm.at[pl.ds(base + w * _CWIN, _CWIN)], sall, isem)
        pltpu.sync_copy(rcv_hbm.at[pl.ds(base + w * _CWIN, _CWIN)], rall)
        cps.wait()
        issue_loads(w, 0, 0)

        def pair(k, carry):
            j0 = 2 * k
            issue_loads(w, j0 + 1, 1)
            drain_loads(0)

            @pl.when((k > 0) | (w > 0))
            def _():
                drain_stores(0)
            compute(0)
            issue_stores(w, j0, 0)

            @pl.when(k < _WPAIR - 1)
            def _():
                issue_loads(w, j0 + 2, 0)
            drain_loads(1)

            @pl.when((k > 0) | (w > 0))
            def _():
                drain_stores(1)
            compute(1)
            issue_stores(w, j0 + 1, 1)
            return carry
        lax.fori_loop(0, _WPAIR, pair, 0)
        return carry
    lax.fori_loop(0, _NWIN, window, 0)

    drain_stores(0)
    drain_stores(1)
    plsc.subcore_barrier()

    @pl.when(s == 0)
    def _writeout():
        pltpu.sync_copy(s_sh, sout_hbm.at[c])


def _sc_edge(first, m, xs, xr, snd, rcv, szero):
    mesh = plsc.VectorSubcoreMesh(core_axis_name="c", subcore_axis_name="s")
    outs = [jax.ShapeDtypeStruct((_NC, _N, _D), jnp.float32)]
    if first:
        outs = [jax.ShapeDtypeStruct((_E, _D), jnp.float32)] + outs
    bufset = [
        pltpu.VMEM((_CHD,), jnp.int32),
        pltpu.VMEM((_CHD, _D), jnp.float32),
        pltpu.VMEM((_CHD, _D), jnp.float32),
        pltpu.VMEM((_CHD, _D), jnp.float32),
        pltpu.VMEM((_CHD, _D), jnp.float32),
    ]
    semset = [pltpu.SemaphoreType.DMA] * 5
    scratch = [pltpu.VMEM((_CWIN,), jnp.int32),
               pltpu.VMEM((_CWIN,), jnp.int32)] \
        + bufset + bufset + semset + semset + [
        pltpu.VMEM_SHARED((_N, _D), jnp.float32),
        pltpu.SemaphoreType.DMA,
    ]
    fn = pl.kernel(
        functools.partial(_sc_body, first),
        out_type=outs,
        mesh=mesh,
        scratch_types=scratch,
    )
    return fn(m, xs, xr, snd, rcv, szero)


def kernel(nodes, edges, senders, receivers,
           l0_We1, l0_be1, l0_We2, l0_be2, l0_Wn1, l0_bn1, l0_Wn2, l0_bn2,
           l1_We1, l1_be1, l1_We2, l1_be2, l1_Wn1, l1_bn1, l1_Wn2, l1_bn2):
    snd = senders.astype(jnp.int32)
    rcv = receivers.astype(jnp.int32)

    we1e0, we1s0, we1r0 = l0_We1[:_D], l0_We1[_D:2 * _D], l0_We1[2 * _D:]
    we1e1, we1s1, we1r1 = l1_We1[:_D], l1_We1[_D:2 * _D], l1_We1[2 * _D:]
    wn1a0, wn1b0 = l0_Wn1[:_D], l0_Wn1[_D:]
    wn1a1, wn1b1 = l1_Wn1[:_D], l1_Wn1[_D:]

    szero = jnp.zeros((_N, _D), jnp.float32)

    wf, c1 = _prep(l0_We2, we1e1, l0_be2, l1_be1)

    cnt2 = _sc_cnt(rcv, szero)

    xs0, xr0 = _nodepre(nodes, we1s0, we1r0, l0_be1)
    m0 = _edge_mm(edges, we1e0)
    h0, s0 = _sc_edge(True, m0, xs0, xr0, snd, rcv, szero)
    nodes1, xs1, xr1 = _node_layer(
        True, nodes, s0[0], s0[1], cnt2[0], cnt2[1],
        l0_We2, l0_be2, wn1a0, wn1b0, l0_bn1, l0_Wn2, l0_bn2,
        we1s1, we1r1, c1)

    m1 = _edge_mm(h0, wf)
    (s1,) = _sc_edge(False, m1, xs1, xr1, snd, rcv, szero)
    (nodes2,) = _node_layer(
        False, nodes1, s1[0], s1[1], cnt2[0], cnt2[1],
        l1_We2, l1_be2, wn1a1, wn1b1, l1_bn1, l1_Wn2, l1_bn2,
        jnp.zeros((_D, _D), jnp.float32), jnp.zeros((_D, _D), jnp.float32),
        c1)
    return nodes2


# speedup vs baseline: 33.6275x; 1.0060x over previous
import functools

import jax
import jax.numpy as jnp
from jax import lax
from jax.experimental import pallas as pl
from jax.experimental.pallas import tpu as pltpu
from jax.experimental.pallas import tpu_sc as plsc

_N = 10000
_E = 320000
_D = 128

_NC = 2
_NS = 16
_NW = _NC * _NS
_EPW = _E // _NW
_CHD = 40
_CWIN = 2000
_NWIN = _EPW // _CWIN
_WPAIR = _CWIN // (2 * _CHD)


def _mm_body(a_ref, w_ref, o_ref):
    o_ref[...] = jnp.dot(a_ref[...], w_ref[...],
                         preferred_element_type=jnp.float32)


def _edge_mm(a, w, be=16000):
    e = a.shape[0]
    return pl.pallas_call(
        _mm_body,
        grid=(e // be,),
        in_specs=[pl.BlockSpec((be, _D), lambda i: (i, 0)),
                  pl.BlockSpec((_D, _D), lambda i: (0, 0))],
        out_specs=pl.BlockSpec((be, _D), lambda i: (i, 0)),
        out_shape=jax.ShapeDtypeStruct((e, _D), jnp.float32),
    )(a, w)


def _prep_body(we2_ref, we1e1_ref, be2_ref, be11_ref, wf_ref, c1_ref):
    wf_ref[...] = jnp.dot(we2_ref[...], we1e1_ref[...],
                          preferred_element_type=jnp.float32)
    c1_ref[...] = be11_ref[...] + jnp.dot(be2_ref[...], we1e1_ref[...],
                                          preferred_element_type=jnp.float32)


def _prep(we2_l0, we1e_l1, be2_l0, be1_l1):
    return pl.pallas_call(
        _prep_body,
        out_shape=[jax.ShapeDtypeStruct((_D, _D), jnp.float32),
                   jax.ShapeDtypeStruct((1, _D), jnp.float32)],
    )(we2_l0, we1e_l1, be2_l0.reshape(1, _D), be1_l1.reshape(1, _D))


def _nodepre_body(x_ref, ws_ref, wr_ref, b_ref, xs_ref, xr_ref):
    x = x_ref[...]
    xs_ref[...] = jnp.dot(x, ws_ref[...],
                          preferred_element_type=jnp.float32) + b_ref[...]
    xr_ref[...] = jnp.dot(x, wr_ref[...], preferred_element_type=jnp.float32)


def _nodepre(nodes, ws, wr, b, bn=1000):
    return pl.pallas_call(
        _nodepre_body,
        grid=(_N // bn,),
        in_specs=[pl.BlockSpec((bn, _D), lambda i: (i, 0)),
                  pl.BlockSpec((_D, _D), lambda i: (0, 0)),
                  pl.BlockSpec((_D, _D), lambda i: (0, 0)),
                  pl.BlockSpec((1, _D), lambda i: (0, 0))],
        out_specs=[pl.BlockSpec((bn, _D), lambda i: (i, 0)),
                   pl.BlockSpec((bn, _D), lambda i: (i, 0))],
        out_shape=[jax.ShapeDtypeStruct((_N, _D), jnp.float32),
                   jax.ShapeDtypeStruct((_N, _D), jnp.float32)],
    )(nodes, ws, wr, b.reshape(1, _D))


def _node_body(first, x_ref, sa_ref, sb_ref, ca_ref, cb_ref,
               we2_ref, be2_ref, wn1a_ref, wn1b_ref, bn1_ref, wn2_ref,
               bn2_ref, ws1_ref, wr1_ref, c1_ref, *out_refs):
    f32 = jnp.float32
    s = sa_ref[...] + sb_ref[...]
    cnt = (ca_ref[...] + cb_ref[...])[:, 0:1]
    eterm = jnp.dot(s, we2_ref[...], preferred_element_type=f32) \
        + cnt * be2_ref[...]
    agg = eterm / jnp.maximum(cnt, 1.0)
    hmid = jnp.maximum(
        jnp.dot(x_ref[...], wn1a_ref[...], preferred_element_type=f32)
        + jnp.dot(agg, wn1b_ref[...], preferred_element_type=f32)
        + bn1_ref[...], 0.0)
    newx = jnp.dot(hmid, wn2_ref[...], preferred_element_type=f32) \
        + bn2_ref[...]
    out_refs[0][...] = newx
    if first:
        out_refs[1][...] = jnp.dot(newx, ws1_ref[...],
                                   preferred_element_type=f32) + c1_ref[...]
        out_refs[2][...] = jnp.dot(newx, wr1_ref[...],
                                   preferred_element_type=f32)


def _node_layer(first, nodes, sa, sb, ca, cb, we2, be2, wn1a, wn1b, bn1,
                wn2, bn2, ws1, wr1, c1, bn=2000):
    blk = lambda r, c: pl.BlockSpec((r, c), lambda i: (i, 0))
    wspec = pl.BlockSpec((_D, _D), lambda i: (0, 0))
    bspec = pl.BlockSpec((1, _D), lambda i: (0, 0))
    nouts = 3 if first else 1
    outs = [jax.ShapeDtypeStruct((_N, _D), jnp.float32)] * nouts
    return pl.pallas_call(
        functools.partial(_node_body, first),
        grid=(_N // bn,),
        in_specs=[blk(bn, _D), blk(bn, _D), blk(bn, _D),
                  blk(bn, _D), blk(bn, _D),
                  wspec, bspec, wspec, wspec, bspec, wspec, bspec,
                  wspec, wspec, bspec],
        out_specs=[blk(bn, _D)] * nouts,
        out_shape=outs,
    )(nodes, sa, sb, ca, cb, we2, be2.reshape(1, _D), wn1a, wn1b,
      bn1.reshape(1, _D), wn2, bn2.reshape(1, _D), ws1, wr1, c1)


def _cnt_body(rcv_hbm, cz_hbm, cout_hbm, rall, ridxs0, ridxs1, ones, c_sh,
              ss0, ss1):
    c = lax.axis_index("c")
    s = lax.axis_index("s")
    wid = s * _NC + c
    base = wid * _EPW
    ridxs = (ridxs0, ridxs1)
    ssem = (ss0, ss1)

    @pl.when(s == 0)
    def _init():
        pltpu.sync_copy(cz_hbm, c_sh)

    lane = lax.iota(jnp.int32, 16)
    onev = jnp.where(lane == 0, 1.0, 0.0).astype(jnp.float32)
    zv = jnp.zeros((16,), jnp.float32)

    def initrow(r, carry):
        for v in range(_D // 16):
            ones[r, pl.ds(v * 16, 16)] = onev if v == 0 else zv
        return carry
    lax.fori_loop(0, _CHD, initrow, 0)
    plsc.subcore_barrier()

    def window(w, carry):
        pltpu.sync_copy(rcv_hbm.at[pl.ds(base + w * _CWIN, _CWIN)], rall)

        def pair(k, carry):
            j0 = 2 * k
            for b in (0, 1):
                o = (j0 + b) * _CHD

                @pl.when((k > 0) | (w > 0))
                def _():
                    pltpu.make_async_copy(ones, c_sh.at[ridxs[b]],
                                          ssem[b]).wait()
                for d in (0, 16, 24):
                    ridxs[b][pl.ds(d, 16)] = rall[pl.ds(o + d, 16)]
                pltpu.async_copy(ones, c_sh.at[ridxs[b]], ssem[b], add=True)
            return carry
        lax.fori_loop(0, _WPAIR, pair, 0)
        return carry
    lax.fori_loop(0, _NWIN, window, 0)

    pltpu.make_async_copy(ones, c_sh.at[ridxs[0]], ssem[0]).wait()
    pltpu.make_async_copy(ones, c_sh.at[ridxs[1]], ssem[1]).wait()
    plsc.subcore_barrier()

    @pl.when(s == 0)
    def _writeout():
        pltpu.sync_copy(c_sh, cout_hbm.at[c])


def _sc_cnt(rcv, czero):
    mesh = plsc.VectorSubcoreMesh(core_axis_name="c", subcore_axis_name="s")
    return pl.kernel(
        _cnt_body,
        out_type=[jax.ShapeDtypeStruct((_NC, _N, _D), jnp.float32)],
        mesh=mesh,
        scratch_types=[
            pltpu.VMEM((_CWIN,), jnp.int32),
            pltpu.VMEM((_CHD,), jnp.int32),
            pltpu.VMEM((_CHD,), jnp.int32),
            pltpu.VMEM((_CHD, _D), jnp.float32),
            pltpu.VMEM_SHARED((_N, _D), jnp.float32),
            pltpu.SemaphoreType.DMA,
            pltpu.SemaphoreType.DMA,
        ],
    )(rcv, czero)[0]


def _sc_body(first, m_hbm, xs_hbm, xr_hbm, snd_hbm, rcv_hbm, sz_hbm, *rest):
    if first:
        h_hbm, sout_hbm = rest[0], rest[1]
        scr = rest[2:]
    else:
        h_hbm = None
        sout_hbm = rest[0]
        scr = rest[1:]
    sall, rall = scr[0], scr[1]
    bufs = (scr[2:7], scr[7:12])
    sems = (scr[12:17], scr[17:22])
    s_sh = scr[22]
    isem = scr[23]

    c = lax.axis_index("c")
    s = lax.axis_index("s")
    wid = s * _NC + c
    base = wid * _EPW

    @pl.when(s == 0)
    def _init():
        pltpu.sync_copy(sz_hbm, s_sh)
    plsc.subcore_barrier()

    def issue_loads(w, t, b):
        (ridxs, xsb, xrb, mb, hb) = bufs[b]
        (gs, gr, gm, ss, sh) = sems[b]
        o = t * _CHD
        pltpu.async_copy(xs_hbm.at[sall.at[pl.ds(o, _CHD)]], xsb, gs)
        pltpu.async_copy(xr_hbm.at[rall.at[pl.ds(o, _CHD)]], xrb, gr)
        pltpu.async_copy(m_hbm.at[pl.ds(base + w * _CWIN + o, _CHD)],
                         mb, gm)

    def drain_loads(b):
        (ridxs, xsb, xrb, mb, hb) = bufs[b]
        (gs, gr, gm, ss, sh) = sems[b]
        pltpu.make_async_copy(xs_hbm.at[sall.at[pl.ds(0, _CHD)]],
                              xsb, gs).wait()
        pltpu.make_async_copy(xr_hbm.at[rall.at[pl.ds(0, _CHD)]],
                              xrb, gr).wait()
        pltpu.make_async_copy(m_hbm.at[pl.ds(0, _CHD)], mb, gm).wait()

    def drain_stores(b):
        (ridxs, xsb, xrb, mb, hb) = bufs[b]
        (gs, gr, gm, ss, sh) = sems[b]
        pltpu.make_async_copy(hb, s_sh.at[ridxs], ss).wait()
        if first:
            pltpu.make_async_copy(hb, h_hbm.at[pl.ds(0, _CHD)], sh).wait()

    def compute(b):
        (ridxs, xsb, xrb, mb, hb) = bufs[b]

        def row(r, carry):
            for v in range(_D // 16):
                sl = pl.ds(v * 16, 16)
                hv = jnp.maximum(mb[r, sl] + xsb[r, sl] + xrb[r, sl], 0.0)
                hb[r, sl] = hv
            return carry
        lax.fori_loop(0, _CHD, row, 0)

    def issue_stores(w, t, b):
        (ridxs, xsb, xrb, mb, hb) = bufs[b]
        (gs, gr, gm, ss, sh) = sems[b]
        o = t * _CHD
        for d in (0, 16, 24):
            ridxs[pl.ds(d, 16)] = rall[pl.ds(o + d, 16)]
        pltpu.async_copy(hb, s_sh.at[ridxs], ss, add=True)
        if first:
            pltpu.async_copy(hb, h_hbm.at[pl.ds(base + w * _CWIN + o, _CHD)],
                             sh)

    def window(w, carry):
        cps = pltpu.async_copy(
            snd_hbm.at[pl.ds(base + w * _CWIN, _CWIN)], sall, isem)
        pltpu.sync_copy(rcv_hbm.at[pl.ds(base + w * _CWIN, _CWIN)], rall)
        cps.wait()
        issue_loads(w, 0, 0)

        def pair(k, carry):
            j0 = 2 * k
            issue_loads(w, j0 + 1, 1)
            drain_loads(0)

            @pl.when((k > 0) | (w > 0))
            def _():
                drain_stores(0)
            compute(0)
            issue_stores(w, j0, 0)

            @pl.when(k < _WPAIR - 1)
            def _():
                issue_loads(w, j0 + 2, 0)
            drain_loads(1)

            @pl.when((k > 0) | (w > 0))
            def _():
                drain_stores(1)
            compute(1)
            issue_stores(w, j0 + 1, 1)
            return carry
        lax.fori_loop(0, _WPAIR, pair, 0)
        return carry
    lax.fori_loop(0, _NWIN, window, 0)

    drain_stores(0)
    drain_stores(1)
    plsc.subcore_barrier()

    @pl.when(s == 0)
    def _writeout():
        pltpu.sync_copy(s_sh, sout_hbm.at[c])


def _sc_edge(first, m, xs, xr, snd, rcv, szero):
    mesh = plsc.VectorSubcoreMesh(core_axis_name="c", subcore_axis_name="s")
    outs = [jax.ShapeDtypeStruct((_NC, _N, _D), jnp.float32)]
    if first:
        outs = [jax.ShapeDtypeStruct((_E, _D), jnp.float32)] + outs
    bufset = [
        pltpu.VMEM((_CHD,), jnp.int32),
        pltpu.VMEM((_CHD, _D), jnp.float32),
        pltpu.VMEM((_CHD, _D), jnp.float32),
        pltpu.VMEM((_CHD, _D), jnp.float32),
        pltpu.VMEM((_CHD, _D), jnp.float32),
    ]
    semset = [pltpu.SemaphoreType.DMA] * 5
    scratch = [pltpu.VMEM((_CWIN,), jnp.int32),
               pltpu.VMEM((_CWIN,), jnp.int32)] \
        + bufset + bufset + semset + semset + [
        pltpu.VMEM_SHARED((_N, _D), jnp.float32),
        pltpu.SemaphoreType.DMA,
    ]
    fn = pl.kernel(
        functools.partial(_sc_body, first),
        out_type=outs,
        mesh=mesh,
        scratch_types=scratch,
    )
    return fn(m, xs, xr, snd, rcv, szero)


def kernel(nodes, edges, senders, receivers,
           l0_We1, l0_be1, l0_We2, l0_be2, l0_Wn1, l0_bn1, l0_Wn2, l0_bn2,
           l1_We1, l1_be1, l1_We2, l1_be2, l1_Wn1, l1_bn1, l1_Wn2, l1_bn2):
    snd = senders.astype(jnp.int32)
    rcv = receivers.astype(jnp.int32)

    we1e0, we1s0, we1r0 = l0_We1[:_D], l0_We1[_D:2 * _D], l0_We1[2 * _D:]
    we1e1, we1s1, we1r1 = l1_We1[:_D], l1_We1[_D:2 * _D], l1_We1[2 * _D:]
    wn1a0, wn1b0 = l0_Wn1[:_D], l0_Wn1[_D:]
    wn1a1, wn1b1 = l1_Wn1[:_D], l1_Wn1[_D:]

    szero = jnp.zeros((_N, _D), jnp.float32)

    wf, c1 = _prep(l0_We2, we1e1, l0_be2, l1_be1)

    cnt2 = _sc_cnt(rcv, szero)

    xs0, xr0 = _nodepre(nodes, we1s0, we1r0, l0_be1)
    m0 = _edge_mm(edges, we1e0)
    h0, s0 = _sc_edge(True, m0, xs0, xr0, snd, rcv, szero)
    nodes1, xs1, xr1 = _node_layer(
        True, nodes, s0[0], s0[1], cnt2[0], cnt2[1],
        l0_We2, l0_be2, wn1a0, wn1b0, l0_bn1, l0_Wn2, l0_bn2,
        we1s1, we1r1, c1)

    m1 = _edge_mm(h0, wf)
    (s1,) = _sc_edge(False, m1, xs1, xr1, snd, rcv, szero)
    (nodes2,) = _node_layer(
        False, nodes1, s1[0], s1[1], cnt2[0], cnt2[1],
        l1_We2, l1_be2, wn1a1, wn1b1, l1_bn1, l1_Wn2, l1_bn2,
        jnp.zeros((_D, _D), jnp.float32), jnp.zeros((_D, _D), jnp.float32),
        c1)
    return nodes2


# speedup vs baseline: 33.7163x; 1.0026x over previous
import functools

import jax
import jax.numpy as jnp
from jax import lax
from jax.experimental import pallas as pl
from jax.experimental.pallas import tpu as pltpu
from jax.experimental.pallas import tpu_sc as plsc

_N = 10000
_E = 320000
_D = 128

_NC = 2
_NS = 16
_NW = _NC * _NS
_EPW = _E // _NW
_CHD = 40
_CWIN = 2000
_NWIN = _EPW // _CWIN
_WPAIR = _CWIN // (2 * _CHD)


def _mm_body(a_ref, w_ref, o_ref):
    o_ref[...] = jnp.dot(a_ref[...], w_ref[...],
                         preferred_element_type=jnp.float32)


def _edge_mm(a, w, be=20000):
    e = a.shape[0]
    return pl.pallas_call(
        _mm_body,
        grid=(e // be,),
        in_specs=[pl.BlockSpec((be, _D), lambda i: (i, 0)),
                  pl.BlockSpec((_D, _D), lambda i: (0, 0))],
        out_specs=pl.BlockSpec((be, _D), lambda i: (i, 0)),
        out_shape=jax.ShapeDtypeStruct((e, _D), jnp.float32),
    )(a, w)


def _prep_body(we2_ref, we1e1_ref, be2_ref, be11_ref, wf_ref, c1_ref):
    wf_ref[...] = jnp.dot(we2_ref[...], we1e1_ref[...],
                          preferred_element_type=jnp.float32)
    c1_ref[...] = be11_ref[...] + jnp.dot(be2_ref[...], we1e1_ref[...],
                                          preferred_element_type=jnp.float32)


def _prep(we2_l0, we1e_l1, be2_l0, be1_l1):
    return pl.pallas_call(
        _prep_body,
        out_shape=[jax.ShapeDtypeStruct((_D, _D), jnp.float32),
                   jax.ShapeDtypeStruct((1, _D), jnp.float32)],
    )(we2_l0, we1e_l1, be2_l0.reshape(1, _D), be1_l1.reshape(1, _D))


def _nodepre_body(x_ref, ws_ref, wr_ref, b_ref, xs_ref, xr_ref):
    x = x_ref[...]
    xs_ref[...] = jnp.dot(x, ws_ref[...],
                          preferred_element_type=jnp.float32) + b_ref[...]
    xr_ref[...] = jnp.dot(x, wr_ref[...], preferred_element_type=jnp.float32)


def _nodepre(nodes, ws, wr, b, bn=1000):
    return pl.pallas_call(
        _nodepre_body,
        grid=(_N // bn,),
        in_specs=[pl.BlockSpec((bn, _D), lambda i: (i, 0)),
                  pl.BlockSpec((_D, _D), lambda i: (0, 0)),
                  pl.BlockSpec((_D, _D), lambda i: (0, 0)),
                  pl.BlockSpec((1, _D), lambda i: (0, 0))],
        out_specs=[pl.BlockSpec((bn, _D), lambda i: (i, 0)),
                   pl.BlockSpec((bn, _D), lambda i: (i, 0))],
        out_shape=[jax.ShapeDtypeStruct((_N, _D), jnp.float32),
                   jax.ShapeDtypeStruct((_N, _D), jnp.float32)],
    )(nodes, ws, wr, b.reshape(1, _D))


def _node_body(first, x_ref, sa_ref, sb_ref, ca_ref, cb_ref,
               we2_ref, be2_ref, wn1a_ref, wn1b_ref, bn1_ref, wn2_ref,
               bn2_ref, ws1_ref, wr1_ref, c1_ref, *out_refs):
    f32 = jnp.float32
    s = sa_ref[...] + sb_ref[...]
    cnt = (ca_ref[...] + cb_ref[...])[:, 0:1]
    eterm = jnp.dot(s, we2_ref[...], preferred_element_type=f32) \
        + cnt * be2_ref[...]
    agg = eterm / jnp.maximum(cnt, 1.0)
    hmid = jnp.maximum(
        jnp.dot(x_ref[...], wn1a_ref[...], preferred_element_type=f32)
        + jnp.dot(agg, wn1b_ref[...], preferred_element_type=f32)
        + bn1_ref[...], 0.0)
    newx = jnp.dot(hmid, wn2_ref[...], preferred_element_type=f32) \
        + bn2_ref[...]
    out_refs[0][...] = newx
    if first:
        out_refs[1][...] = jnp.dot(newx, ws1_ref[...],
                                   preferred_element_type=f32) + c1_ref[...]
        out_refs[2][...] = jnp.dot(newx, wr1_ref[...],
                                   preferred_element_type=f32)


def _node_layer(first, nodes, sa, sb, ca, cb, we2, be2, wn1a, wn1b, bn1,
                wn2, bn2, ws1, wr1, c1, bn=2000):
    blk = lambda r, c: pl.BlockSpec((r, c), lambda i: (i, 0))
    wspec = pl.BlockSpec((_D, _D), lambda i: (0, 0))
    bspec = pl.BlockSpec((1, _D), lambda i: (0, 0))
    nouts = 3 if first else 1
    outs = [jax.ShapeDtypeStruct((_N, _D), jnp.float32)] * nouts
    return pl.pallas_call(
        functools.partial(_node_body, first),
        grid=(_N // bn,),
        in_specs=[blk(bn, _D), blk(bn, _D), blk(bn, _D),
                  blk(bn, _D), blk(bn, _D),
                  wspec, bspec, wspec, wspec, bspec, wspec, bspec,
                  wspec, wspec, bspec],
        out_specs=[blk(bn, _D)] * nouts,
        out_shape=outs,
    )(nodes, sa, sb, ca, cb, we2, be2.reshape(1, _D), wn1a, wn1b,
      bn1.reshape(1, _D), wn2, bn2.reshape(1, _D), ws1, wr1, c1)


def _cnt_body(rcv_hbm, cz_hbm, cout_hbm, rall, ridxs0, ridxs1, ones, c_sh,
              ss0, ss1):
    c = lax.axis_index("c")
    s = lax.axis_index("s")
    wid = s * _NC + c
    base = wid * _EPW
    ridxs = (ridxs0, ridxs1)
    ssem = (ss0, ss1)

    @pl.when(s == 0)
    def _init():
        pltpu.sync_copy(cz_hbm, c_sh)

    lane = lax.iota(jnp.int32, 16)
    onev = jnp.where(lane == 0, 1.0, 0.0).astype(jnp.float32)
    zv = jnp.zeros((16,), jnp.float32)

    def initrow(r, carry):
        for v in range(_D // 16):
            ones[r, pl.ds(v * 16, 16)] = onev if v == 0 else zv
        return carry
    lax.fori_loop(0, _CHD, initrow, 0)
    plsc.subcore_barrier()

    def window(w, carry):
        pltpu.sync_copy(rcv_hbm.at[pl.ds(base + w * _CWIN, _CWIN)], rall)

        def pair(k, carry):
            j0 = 2 * k
            for b in (0, 1):
                o = (j0 + b) * _CHD

                @pl.when((k > 0) | (w > 0))
                def _():
                    pltpu.make_async_copy(ones, c_sh.at[ridxs[b]],
                                          ssem[b]).wait()
                for d in (0, 16, 24):
                    ridxs[b][pl.ds(d, 16)] = rall[pl.ds(o + d, 16)]
                pltpu.async_copy(ones, c_sh.at[ridxs[b]], ssem[b], add=True)
            return carry
        lax.fori_loop(0, _WPAIR, pair, 0)
        return carry
    lax.fori_loop(0, _NWIN, window, 0)

    pltpu.make_async_copy(ones, c_sh.at[ridxs[0]], ssem[0]).wait()
    pltpu.make_async_copy(ones, c_sh.at[ridxs[1]], ssem[1]).wait()
    plsc.subcore_barrier()

    @pl.when(s == 0)
    def _writeout():
        pltpu.sync_copy(c_sh, cout_hbm.at[c])


def _sc_cnt(rcv, czero):
    mesh = plsc.VectorSubcoreMesh(core_axis_name="c", subcore_axis_name="s")
    return pl.kernel(
        _cnt_body,
        out_type=[jax.ShapeDtypeStruct((_NC, _N, _D), jnp.float32)],
        mesh=mesh,
        scratch_types=[
            pltpu.VMEM((_CWIN,), jnp.int32),
            pltpu.VMEM((_CHD,), jnp.int32),
            pltpu.VMEM((_CHD,), jnp.int32),
            pltpu.VMEM((_CHD, _D), jnp.float32),
            pltpu.VMEM_SHARED((_N, _D), jnp.float32),
            pltpu.SemaphoreType.DMA,
            pltpu.SemaphoreType.DMA,
        ],
    )(rcv, czero)[0]


def _sc_body(first, m_hbm, xs_hbm, xr_hbm, snd_hbm, rcv_hbm, sz_hbm, *rest):
    if first:
        h_hbm, sout_hbm = rest[0], rest[1]
        scr = rest[2:]
    else:
        h_hbm = None
        sout_hbm = rest[0]
        scr = rest[1:]
    sall, rall = scr[0], scr[1]
    bufs = (scr[2:7], scr[7:12])
    sems = (scr[12:17], scr[17:22])
    s_sh = scr[22]
    isem = scr[23]

    c = lax.axis_index("c")
    s = lax.axis_index("s")
    wid = s * _NC + c
    base = wid * _EPW

    @pl.when(s == 0)
    def _init():
        pltpu.sync_copy(sz_hbm, s_sh)
    plsc.subcore_barrier()

    def issue_loads(w, t, b):
        (ridxs, xsb, xrb, mb, hb) = bufs[b]
        (gs, gr, gm, ss, sh) = sems[b]
        o = t * _CHD
        pltpu.async_copy(xs_hbm.at[sall.at[pl.ds(o, _CHD)]], xsb, gs)
        pltpu.async_copy(xr_hbm.at[rall.at[pl.ds(o, _CHD)]], xrb, gr)
        pltpu.async_copy(m_hbm.at[pl.ds(base + w * _CWIN + o, _CHD)],
                         mb, gm)

    def drain_loads(b):
        (ridxs, xsb, xrb, mb, hb) = bufs[b]
        (gs, gr, gm, ss, sh) = sems[b]
        pltpu.make_async_copy(xs_hbm.at[sall.at[pl.ds(0, _CHD)]],
                              xsb, gs).wait()
        pltpu.make_async_copy(xr_hbm.at[rall.at[pl.ds(0, _CHD)]],
                              xrb, gr).wait()
        pltpu.make_async_copy(m_hbm.at[pl.ds(0, _CHD)], mb, gm).wait()

    def drain_stores(b):
        (ridxs, xsb, xrb, mb, hb) = bufs[b]
        (gs, gr, gm, ss, sh) = sems[b]
        pltpu.make_async_copy(hb, s_sh.at[ridxs], ss).wait()
        if first:
            pltpu.make_async_copy(hb, h_hbm.at[pl.ds(0, _CHD)], sh).wait()

    def compute(b):
        (ridxs, xsb, xrb, mb, hb) = bufs[b]

        def row(r, carry):
            for v in range(_D // 16):
                sl = pl.ds(v * 16, 16)
                hv = jnp.maximum(mb[r, sl] + xsb[r, sl] + xrb[r, sl], 0.0)
                hb[r, sl] = hv
            return carry
        lax.fori_loop(0, _CHD, row, 0)

    def issue_stores(w, t, b):
        (ridxs, xsb, xrb, mb, hb) = bufs[b]
        (gs, gr, gm, ss, sh) = sems[b]
        o = t * _CHD
        for d in (0, 16, 24):
            ridxs[pl.ds(d, 16)] = rall[pl.ds(o + d, 16)]
        pltpu.async_copy(hb, s_sh.at[ridxs], ss, add=True)
        if first:
            pltpu.async_copy(hb, h_hbm.at[pl.ds(base + w * _CWIN + o, _CHD)],
                             sh)

    def window(w, carry):
        cps = pltpu.async_copy(
            snd_hbm.at[pl.ds(base + w * _CWIN, _CWIN)], sall, isem)
        pltpu.sync_copy(rcv_hbm.at[pl.ds(base + w * _CWIN, _CWIN)], rall)
        cps.wait()
        issue_loads(w, 0, 0)

        def pair(k, carry):
            j0 = 2 * k
            issue_loads(w, j0 + 1, 1)
            drain_loads(0)

            @pl.when((k > 0) | (w > 0))
            def _():
                drain_stores(0)
            compute(0)
            issue_stores(w, j0, 0)

            @pl.when(k < _WPAIR - 1)
            def _():
                issue_loads(w, j0 + 2, 0)
            drain_loads(1)

            @pl.when((k > 0) | (w > 0))
            def _():
                drain_stores(1)
            compute(1)
            issue_stores(w, j0 + 1, 1)
            return carry
        lax.fori_loop(0, _WPAIR, pair, 0)
        return carry
    lax.fori_loop(0, _NWIN, window, 0)

    drain_stores(0)
    drain_stores(1)
    plsc.subcore_barrier()

    @pl.when(s == 0)
    def _writeout():
        pltpu.sync_copy(s_sh, sout_hbm.at[c])


def _sc_edge(first, m, xs, xr, snd, rcv, szero):
    mesh = plsc.VectorSubcoreMesh(core_axis_name="c", subcore_axis_name="s")
    outs = [jax.ShapeDtypeStruct((_NC, _N, _D), jnp.float32)]
    if first:
        outs = [jax.ShapeDtypeStruct((_E, _D), jnp.float32)] + outs
    bufset = [
        pltpu.VMEM((_CHD,), jnp.int32),
        pltpu.VMEM((_CHD, _D), jnp.float32),
        pltpu.VMEM((_CHD, _D), jnp.float32),
        pltpu.VMEM((_CHD, _D), jnp.float32),
        pltpu.VMEM((_CHD, _D), jnp.float32),
    ]
    semset = [pltpu.SemaphoreType.DMA] * 5
    scratch = [pltpu.VMEM((_CWIN,), jnp.int32),
               pltpu.VMEM((_CWIN,), jnp.int32)] \
        + bufset + bufset + semset + semset + [
        pltpu.VMEM_SHARED((_N, _D), jnp.float32),
        pltpu.SemaphoreType.DMA,
    ]
    fn = pl.kernel(
        functools.partial(_sc_body, first),
        out_type=outs,
        mesh=mesh,
        scratch_types=scratch,
    )
    return fn(m, xs, xr, snd, rcv, szero)


def kernel(nodes, edges, senders, receivers,
           l0_We1, l0_be1, l0_We2, l0_be2, l0_Wn1, l0_bn1, l0_Wn2, l0_bn2,
           l1_We1, l1_be1, l1_We2, l1_be2, l1_Wn1, l1_bn1, l1_Wn2, l1_bn2):
    snd = senders.astype(jnp.int32)
    rcv = receivers.astype(jnp.int32)

    we1e0, we1s0, we1r0 = l0_We1[:_D], l0_We1[_D:2 * _D], l0_We1[2 * _D:]
    we1e1, we1s1, we1r1 = l1_We1[:_D], l1_We1[_D:2 * _D], l1_We1[2 * _D:]
    wn1a0, wn1b0 = l0_Wn1[:_D], l0_Wn1[_D:]
    wn1a1, wn1b1 = l1_Wn1[:_D], l1_Wn1[_D:]

    szero = jnp.zeros((_N, _D), jnp.float32)

    wf, c1 = _prep(l0_We2, we1e1, l0_be2, l1_be1)

    cnt2 = _sc_cnt(rcv, szero)

    xs0, xr0 = _nodepre(nodes, we1s0, we1r0, l0_be1)
    m0 = _edge_mm(edges, we1e0)
    h0, s0 = _sc_edge(True, m0, xs0, xr0, snd, rcv, szero)
    nodes1, xs1, xr1 = _node_layer(
        True, nodes, s0[0], s0[1], cnt2[0], cnt2[1],
        l0_We2, l0_be2, wn1a0, wn1b0, l0_bn1, l0_Wn2, l0_bn2,
        we1s1, we1r1, c1)

    m1 = _edge_mm(h0, wf)
    (s1,) = _sc_edge(False, m1, xs1, xr1, snd, rcv, szero)
    (nodes2,) = _node_layer(
        False, nodes1, s1[0], s1[1], cnt2[0], cnt2[1],
        l1_We2, l1_be2, wn1a1, wn1b1, l1_bn1, l1_Wn2, l1_bn2,
        jnp.zeros((_D, _D), jnp.float32), jnp.zeros((_D, _D), jnp.float32),
        c1)
    return nodes2
